# baseline (speedup 1.0000x reference)
import functools

import jax
import jax.numpy as jnp
from jax import lax
from jax.experimental import pallas as pl
from jax.experimental.pallas import tpu as pltpu
from jax.experimental.pallas import tpu_sc as plsc

D = 1024
D2 = 512
M = 8
F = 4096
N = 4096
CAP = 640
TB = 1024
FBR = 2048
NTB = N // TB
NFBR = F // FBR
FBM = 2048
NFBM = F // FBM
SB = 512


def _gelu_exact(v):
    return v * (0.5 * (1.0 + lax.erf(v * 0.7071067811865476)))


NC, NS = 2, 16
NW = NC * NS
TOK_PER_W = N // NW
SUB = 32
NSUB = TOK_PER_W // SUB


def _router_kernel(x_ref, w1_ref, b1_ref, w2_ref, b2_ref,
                   logits_ref, dest_ref, safe_ref, maskf_ref,
                   acc_ref, cnt_ref):
    tb = pl.program_id(0)
    fb = pl.program_id(1)

    @pl.when(jnp.logical_and(tb == 0, fb == 0))
    def _init():
        cnt_ref[...] = jnp.zeros_like(cnt_ref)

    h = lax.dot_general(x_ref[...], w1_ref[...], (((1,), (1,)), ((), ())),
                        preferred_element_type=jnp.float32)
    h = _gelu_exact(h + b1_ref[...])
    part = lax.dot_general(h, w2_ref[...], (((1,), (1,)), ((), ())),
                           preferred_element_type=jnp.float32)

    @pl.when(fb == 0)
    def _first():
        acc_ref[...] = part

    @pl.when(fb > 0)
    def _rest():
        acc_ref[...] += part

    @pl.when(fb == NFBR - 1)
    def _final():
        logits = acc_ref[...] + b2_ref[...]
        logits_ref[...] = logits
        mx = jnp.max(logits, axis=1, keepdims=True)
        lane = lax.broadcasted_iota(jnp.int32, (TB, M), 1)
        idx = jnp.min(jnp.where(logits == mx, lane, M), axis=1,
                      keepdims=True)
        oh = (lane == idx).astype(jnp.float32)
        row = lax.broadcasted_iota(jnp.int32, (TB, TB), 0)
        col = lax.broadcasted_iota(jnp.int32, (TB, TB), 1)
        tril = (row >= col).astype(jnp.float32)
        counts = lax.dot_general(tril, oh, (((1,), (0,)), ((), ())),
                                 preferred_element_type=jnp.float32)
        counts = counts + cnt_ref[...]
        pos = jnp.sum(counts * oh, axis=1, keepdims=True) - 1.0
        posi = pos.astype(jnp.int32)
        maskb = posi < CAP
        d = idx * CAP + posi
        dest_ref[...] = jnp.where(maskb, d, M * CAP)
        safe_ref[...] = jnp.where(maskb, d, 0)
        maskf_ref[...] = maskb.astype(jnp.float32)
        cnt_ref[...] += jnp.sum(oh, axis=0, keepdims=True)


def _router_call(x_flat, w1_w, w1_b, w2_w, w2_b):
    return pl.pallas_call(
        _router_kernel,
        grid=(NTB, NFBR),
        in_specs=[
            pl.BlockSpec((TB, D), lambda tb, fb: (tb, 0)),
            pl.BlockSpec((FBR, D), lambda tb, fb: (fb, 0)),
            pl.BlockSpec((1, FBR), lambda tb, fb: (0, fb)),
            pl.BlockSpec((M, FBR), lambda tb, fb: (0, fb)),
            pl.BlockSpec((1, M), lambda tb, fb: (0, 0)),
        ],
        out_specs=[
            pl.BlockSpec((TB, M), lambda tb, fb: (tb, 0)),
            pl.BlockSpec((TB, 1), lambda tb, fb: (tb, 0)),
            pl.BlockSpec((TB, 1), lambda tb, fb: (tb, 0)),
            pl.BlockSpec((TB, 1), lambda tb, fb: (tb, 0)),
        ],
        out_shape=[
            jax.ShapeDtypeStruct((N, M), jnp.float32),
            jax.ShapeDtypeStruct((N, 1), jnp.int32),
            jax.ShapeDtypeStruct((N, 1), jnp.int32),
            jax.ShapeDtypeStruct((N, 1), jnp.float32),
        ],
        scratch_shapes=[
            pltpu.VMEM((TB, M), jnp.float32),
            pltpu.VMEM((1, M), jnp.float32),
        ],
    )(x_flat, w1_w, w1_b, w2_w, w2_b)


@functools.lru_cache(maxsize=None)
def _dispatch_fn():
    mesh = plsc.VectorSubcoreMesh(core_axis_name="c", subcore_axis_name="s")

    @functools.partial(
        pl.kernel,
        mesh=mesh,
        out_type=jax.ShapeDtypeStruct((M * CAP + 1, D2), jnp.int32),
        scratch_types=(
            [pltpu.VMEM((SUB,), jnp.int32) for _ in range(NSUB)]
            + [pltpu.VMEM((SUB, D2), jnp.int32) for _ in range(3)]
            + [pltpu.SemaphoreType.DMA for _ in range(7)]
        ),
    )
    def dispatch(x_hbm, dest_hbm, ebuf_hbm,
                 i0, i1, i2, i3, r0, r1, r2,
                 isem, l0, l1, l2, s0, s1, s2):
        wid = lax.axis_index("s") * NC + lax.axis_index("c")
        base = wid * TOK_PER_W
        idxs = (i0, i1, i2, i3)
        bufs = (r0, r1, r2)
        lsems = (l0, l1, l2)
        ssems = (s0, s1, s2)
        ih = [pltpu.async_copy(dest_hbm.at[pl.ds(base + j * SUB, SUB)],
                               idxs[j], isem) for j in range(NSUB)]
        lh = [None] * NSUB
        sh = [None] * NSUB
        for j in range(3):
            lh[j] = pltpu.async_copy(x_hbm.at[pl.ds(base + j * SUB, SUB)],
                                     bufs[j], lsems[j])
        for h in ih:
            h.wait()
        waited = [False] * NSUB
        for j in range(NSUB):
            lh[j].wait()
            sh[j] = pltpu.async_copy(bufs[j % 3], ebuf_hbm.at[idxs[j]],
                                     ssems[j % 3])
            if j >= 1 and j + 2 < NSUB:
                sh[j - 1].wait()
                waited[j - 1] = True
                lh[j + 2] = pltpu.async_copy(
                    x_hbm.at[pl.ds(base + (j + 2) * SUB, SUB)],
                    bufs[(j + 2) % 3], lsems[(j + 2) % 3])
        for j in range(NSUB):
            if not waited[j]:
                sh[j].wait()

    return dispatch


def _mlp_kernel(ei_ref, m1_ref, m2_ref, out_ref, acc_ref):
    fb = pl.program_id(1)
    ei = ei_ref[...].astype(jnp.float32)
    h = lax.dot_general(ei, m1_ref[0], (((1,), (0,)), ((), ())),
                        preferred_element_type=jnp.float32)
    h = _gelu_exact(h)
    part = lax.dot_general(h, m2_ref[0], (((1,), (0,)), ((), ())),
                           preferred_element_type=jnp.float32)

    @pl.when(fb == 0)
    def _first():
        acc_ref[...] = part

    @pl.when(fb > 0)
    def _rest():
        acc_ref[...] += part

    @pl.when(fb == NFBM - 1)
    def _final():
        out_ref[...] = acc_ref[...].astype(jnp.bfloat16)


def _mlp_call(ebuf, mlp1, mlp2):
    return pl.pallas_call(
        _mlp_kernel,
        grid=(M, NFBM),
        in_specs=[
            pl.BlockSpec((CAP, D), lambda e, fb: (e, 0)),
            pl.BlockSpec((1, D, FBM), lambda e, fb: (e, 0, fb)),
            pl.BlockSpec((1, FBM, D), lambda e, fb: (e, fb, 0)),
        ],
        out_specs=pl.BlockSpec((CAP, D), lambda e, fb: (e, 0)),
        out_shape=jax.ShapeDtypeStruct((M * CAP, D), jnp.bfloat16),
        scratch_shapes=[pltpu.VMEM((CAP, D), jnp.float32)],
    )(ebuf, mlp1, mlp2)


@functools.lru_cache(maxsize=None)
def _combine_fn():
    mesh = plsc.VectorSubcoreMesh(core_axis_name="c", subcore_axis_name="s")

    @functools.partial(
        pl.kernel,
        mesh=mesh,
        out_type=jax.ShapeDtypeStruct((N, D2), jnp.int32),
        scratch_types=(
            [pltpu.VMEM((SUB,), jnp.int32) for _ in range(NSUB)]
            + [pltpu.VMEM((SUB, D2), jnp.int32) for _ in range(3)]
            + [pltpu.SemaphoreType.DMA for _ in range(7)]
        ),
    )
    def combine(tab_hbm, safe_hbm, out_hbm,
                i0, i1, i2, i3, r0, r1, r2,
                isem, g0, g1, g2, s0, s1, s2):
        wid = lax.axis_index("s") * NC + lax.axis_index("c")
        base = wid * TOK_PER_W
        idxs = (i0, i1, i2, i3)
        bufs = (r0, r1, r2)
        gsems = (g0, g1, g2)
        ssems = (s0, s1, s2)
        ih = [pltpu.async_copy(safe_hbm.at[pl.ds(base + j * SUB, SUB)],
                               idxs[j], isem) for j in range(NSUB)]
        for h in ih:
            h.wait()
        gh = [None] * NSUB
        sh = [None] * NSUB
        for j in range(3):
            gh[j] = pltpu.async_copy(tab_hbm.at[idxs[j]], bufs[j], gsems[j])
        waited = [False] * NSUB
        for j in range(NSUB):
            gh[j].wait()
            sh[j] = pltpu.async_copy(bufs[j % 3],
                                     out_hbm.at[pl.ds(base + j * SUB, SUB)],
                                     ssems[j % 3])
            if j >= 1 and j + 2 < NSUB:
                sh[j - 1].wait()
                waited[j - 1] = True
                gh[j + 2] = pltpu.async_copy(tab_hbm.at[idxs[j + 2]],
                                             bufs[(j + 2) % 3],
                                             gsems[(j + 2) % 3])
        for j in range(NSUB):
            if not waited[j]:
                sh[j].wait()

    return combine


def _select_kernel(g_ref, x_ref, m_ref, out_ref):
    out_ref[...] = jnp.where(m_ref[...] > 0.5,
                             g_ref[...].astype(jnp.float32), x_ref[...])


def _select_call(gath, x_flat, maskf):
    return pl.pallas_call(
        _select_kernel,
        grid=(N // SB,),
        in_specs=[
            pl.BlockSpec((SB, D), lambda tb: (tb, 0)),
            pl.BlockSpec((SB, D), lambda tb: (tb, 0)),
            pl.BlockSpec((SB, 1), lambda tb: (tb, 0)),
        ],
        out_specs=pl.BlockSpec((SB, D), lambda tb: (tb, 0)),
        out_shape=jax.ShapeDtypeStruct((N, D), jnp.float32),
    )(gath, x_flat, maskf)


def kernel(x, mlp1, mlp2, w1_w, w1_b, w2_w, w2_b):
    b, s, d = x.shape
    x_flat = x.reshape(b * s, d)
    logits, dest, safe, maskf = _router_call(
        x_flat, w1_w, w1_b.reshape(1, F), w2_w, w2_b.reshape(1, M))
    x_i32 = lax.bitcast_convert_type(
        x_flat.astype(jnp.bfloat16).reshape(N, D2, 2), jnp.int32)
    ebuf_i32 = _dispatch_fn()(x_i32, dest.reshape(N))
    ebuf_bf = lax.bitcast_convert_type(
        ebuf_i32, jnp.bfloat16).reshape(M * CAP + 1, D)
    mlp_out = _mlp_call(ebuf_bf, mlp1, mlp2)
    tab_i32 = lax.bitcast_convert_type(
        mlp_out.reshape(M * CAP, D2, 2), jnp.int32)
    gath_i32 = _combine_fn()(tab_i32, safe.reshape(N))
    gath = lax.bitcast_convert_type(gath_i32, jnp.bfloat16).reshape(N, D)
    out = _select_call(gath, x_flat, maskf)
    return out.reshape(b, s, d), logits.reshape(b, s, M)


# speedup vs baseline: 2.3175x; 2.3175x over previous
import functools

import jax
import jax.numpy as jnp
from jax import lax
from jax.experimental import pallas as pl
from jax.experimental.pallas import tpu as pltpu
from jax.experimental.pallas import tpu_sc as plsc

D = 1024
D2 = 512
M = 8
F = 4096
N = 4096
CAP = 640
TB = 1024
FBR = 2048
NTB = N // TB
NFBR = F // FBR
FBM = 2048
NFBM = F // FBM
SB = 512


def _gelu_exact(v):
    return v * (0.5 * (1.0 + lax.erf(v * 0.7071067811865476)))


NC, NS = 2, 16
NW = NC * NS
TOK_PER_W = N // NW
SUB = 32
NSUB = TOK_PER_W // SUB


def _router_kernel(x_ref, w1_ref, b1_ref, w2_ref, b2_ref,
                   logits_ref, dest_ref, safe_ref, maskf_ref,
                   acc_ref, cnt_ref):
    tb = pl.program_id(0)
    fb = pl.program_id(1)

    @pl.when(jnp.logical_and(tb == 0, fb == 0))
    def _init():
        cnt_ref[...] = jnp.zeros_like(cnt_ref)

    h = lax.dot_general(x_ref[...], w1_ref[...], (((1,), (1,)), ((), ())),
                        preferred_element_type=jnp.float32)
    h = _gelu_exact(h + b1_ref[...])
    part = lax.dot_general(h, w2_ref[...], (((1,), (1,)), ((), ())),
                           preferred_element_type=jnp.float32)

    @pl.when(fb == 0)
    def _first():
        acc_ref[...] = part

    @pl.when(fb > 0)
    def _rest():
        acc_ref[...] += part

    @pl.when(fb == NFBR - 1)
    def _final():
        logits = acc_ref[...] + b2_ref[...]
        logits_ref[...] = logits
        mx = jnp.max(logits, axis=1, keepdims=True)
        lane = lax.broadcasted_iota(jnp.int32, (TB, M), 1)
        idx = jnp.min(jnp.where(logits == mx, lane, M), axis=1,
                      keepdims=True)
        oh = (lane == idx).astype(jnp.float32)
        row = lax.broadcasted_iota(jnp.int32, (TB, TB), 0)
        col = lax.broadcasted_iota(jnp.int32, (TB, TB), 1)
        tril = (row >= col).astype(jnp.float32)
        counts = lax.dot_general(tril, oh, (((1,), (0,)), ((), ())),
                                 preferred_element_type=jnp.float32)
        counts = counts + cnt_ref[...]
        pos = jnp.sum(counts * oh, axis=1, keepdims=True) - 1.0
        posi = pos.astype(jnp.int32)
        maskb = posi < CAP
        d = idx * CAP + posi
        dest_ref[...] = jnp.where(maskb, d, M * CAP)
        safe_ref[...] = jnp.where(maskb, d, 0)
        maskf_ref[...] = maskb.astype(jnp.float32)
        cnt_ref[...] += jnp.sum(oh, axis=0, keepdims=True)


def _router_call(x_flat, w1_w, w1_b, w2_w, w2_b):
    return pl.pallas_call(
        _router_kernel,
        grid=(NTB, NFBR),
        in_specs=[
            pl.BlockSpec((TB, D), lambda tb, fb: (tb, 0)),
            pl.BlockSpec((FBR, D), lambda tb, fb: (fb, 0)),
            pl.BlockSpec((1, FBR), lambda tb, fb: (0, fb)),
            pl.BlockSpec((M, FBR), lambda tb, fb: (0, fb)),
            pl.BlockSpec((1, M), lambda tb, fb: (0, 0)),
        ],
        out_specs=[
            pl.BlockSpec((TB, M), lambda tb, fb: (tb, 0)),
            pl.BlockSpec((TB, 1), lambda tb, fb: (tb, 0)),
            pl.BlockSpec((TB, 1), lambda tb, fb: (tb, 0)),
            pl.BlockSpec((TB, 1), lambda tb, fb: (tb, 0)),
        ],
        out_shape=[
            jax.ShapeDtypeStruct((N, M), jnp.float32),
            jax.ShapeDtypeStruct((N, 1), jnp.int32),
            jax.ShapeDtypeStruct((N, 1), jnp.int32),
            jax.ShapeDtypeStruct((N, 1), jnp.float32),
        ],
        scratch_shapes=[
            pltpu.VMEM((TB, M), jnp.float32),
            pltpu.VMEM((1, M), jnp.float32),
        ],
    )(x_flat, w1_w, w1_b, w2_w, w2_b)


@functools.lru_cache(maxsize=None)
def _dispatch_fn():
    mesh = plsc.VectorSubcoreMesh(core_axis_name="c", subcore_axis_name="s")

    @functools.partial(
        pl.kernel,
        mesh=mesh,
        out_type=jax.ShapeDtypeStruct((M * CAP + 1, D), jnp.float32),
        scratch_types=(
            [pltpu.VMEM((SUB,), jnp.int32) for _ in range(NSUB)]
            + [pltpu.VMEM((SUB, D), jnp.float32) for _ in range(3)]
            + [pltpu.SemaphoreType.DMA for _ in range(7)]
        ),
    )
    def dispatch(x_hbm, dest_hbm, ebuf_hbm,
                 i0, i1, i2, i3, r0, r1, r2,
                 isem, l0, l1, l2, s0, s1, s2):
        wid = lax.axis_index("s") * NC + lax.axis_index("c")
        base = wid * TOK_PER_W
        idxs = (i0, i1, i2, i3)
        bufs = (r0, r1, r2)
        lsems = (l0, l1, l2)
        ssems = (s0, s1, s2)
        ih = [pltpu.async_copy(dest_hbm.at[pl.ds(base + j * SUB, SUB)],
                               idxs[j], isem) for j in range(NSUB)]
        lh = [None] * NSUB
        sh = [None] * NSUB
        for j in range(3):
            lh[j] = pltpu.async_copy(x_hbm.at[pl.ds(base + j * SUB, SUB)],
                                     bufs[j], lsems[j])
        for h in ih:
            h.wait()
        waited = [False] * NSUB
        for j in range(NSUB):
            lh[j].wait()
            sh[j] = pltpu.async_copy(bufs[j % 3], ebuf_hbm.at[idxs[j]],
                                     ssems[j % 3])
            if j >= 1 and j + 2 < NSUB:
                sh[j - 1].wait()
                waited[j - 1] = True
                lh[j + 2] = pltpu.async_copy(
                    x_hbm.at[pl.ds(base + (j + 2) * SUB, SUB)],
                    bufs[(j + 2) % 3], lsems[(j + 2) % 3])
        for j in range(NSUB):
            if not waited[j]:
                sh[j].wait()

    return dispatch


def _mlp_kernel(ei_ref, m1_ref, m2_ref, out_ref, acc_ref):
    fb = pl.program_id(1)
    h = lax.dot_general(ei_ref[...], m1_ref[0], (((1,), (0,)), ((), ())),
                        preferred_element_type=jnp.float32)
    h = _gelu_exact(h)
    part = lax.dot_general(h, m2_ref[0], (((1,), (0,)), ((), ())),
                           preferred_element_type=jnp.float32)

    @pl.when(fb == 0)
    def _first():
        acc_ref[...] = part

    @pl.when(fb > 0)
    def _rest():
        acc_ref[...] += part

    @pl.when(fb == NFBM - 1)
    def _final():
        out_ref[...] = acc_ref[...]


def _mlp_call(ebuf, mlp1, mlp2):
    return pl.pallas_call(
        _mlp_kernel,
        grid=(M, NFBM),
        in_specs=[
            pl.BlockSpec((CAP, D), lambda e, fb: (e, 0)),
            pl.BlockSpec((1, D, FBM), lambda e, fb: (e, 0, fb)),
            pl.BlockSpec((1, FBM, D), lambda e, fb: (e, fb, 0)),
        ],
        out_specs=pl.BlockSpec((CAP, D), lambda e, fb: (e, 0)),
        out_shape=jax.ShapeDtypeStruct((M * CAP, D), jnp.float32),
        scratch_shapes=[pltpu.VMEM((CAP, D), jnp.float32)],
    )(ebuf, mlp1, mlp2)


@functools.lru_cache(maxsize=None)
def _combine_fn():
    mesh = plsc.VectorSubcoreMesh(core_axis_name="c", subcore_axis_name="s")

    @functools.partial(
        pl.kernel,
        mesh=mesh,
        out_type=jax.ShapeDtypeStruct((N, D), jnp.float32),
        scratch_types=(
            [pltpu.VMEM((SUB,), jnp.int32) for _ in range(NSUB)]
            + [pltpu.VMEM((SUB, D), jnp.float32) for _ in range(3)]
            + [pltpu.SemaphoreType.DMA for _ in range(7)]
        ),
    )
    def combine(tab_hbm, safe_hbm, out_hbm,
                i0, i1, i2, i3, r0, r1, r2,
                isem, g0, g1, g2, s0, s1, s2):
        wid = lax.axis_index("s") * NC + lax.axis_index("c")
        base = wid * TOK_PER_W
        idxs = (i0, i1, i2, i3)
        bufs = (r0, r1, r2)
        gsems = (g0, g1, g2)
        ssems = (s0, s1, s2)
        ih = [pltpu.async_copy(safe_hbm.at[pl.ds(base + j * SUB, SUB)],
                               idxs[j], isem) for j in range(NSUB)]
        for h in ih:
            h.wait()
        gh = [None] * NSUB
        sh = [None] * NSUB
        for j in range(3):
            gh[j] = pltpu.async_copy(tab_hbm.at[idxs[j]], bufs[j], gsems[j])
        waited = [False] * NSUB
        for j in range(NSUB):
            gh[j].wait()
            sh[j] = pltpu.async_copy(bufs[j % 3],
                                     out_hbm.at[pl.ds(base + j * SUB, SUB)],
                                     ssems[j % 3])
            if j >= 1 and j + 2 < NSUB:
                sh[j - 1].wait()
                waited[j - 1] = True
                gh[j + 2] = pltpu.async_copy(tab_hbm.at[idxs[j + 2]],
                                             bufs[(j + 2) % 3],
                                             gsems[(j + 2) % 3])
        for j in range(NSUB):
            if not waited[j]:
                sh[j].wait()

    return combine


def _select_kernel(g_ref, x_ref, m_ref, out_ref):
    out_ref[...] = jnp.where(m_ref[...] > 0.5, g_ref[...], x_ref[...])


def _select_call(gath, x_flat, maskf):
    return pl.pallas_call(
        _select_kernel,
        grid=(N // SB,),
        in_specs=[
            pl.BlockSpec((SB, D), lambda tb: (tb, 0)),
            pl.BlockSpec((SB, D), lambda tb: (tb, 0)),
            pl.BlockSpec((SB, 1), lambda tb: (tb, 0)),
        ],
        out_specs=pl.BlockSpec((SB, D), lambda tb: (tb, 0)),
        out_shape=jax.ShapeDtypeStruct((N, D), jnp.float32),
    )(gath, x_flat, maskf)


def kernel(x, mlp1, mlp2, w1_w, w1_b, w2_w, w2_b):
    b, s, d = x.shape
    x_flat = x.reshape(b * s, d)
    logits, dest, safe, maskf = _router_call(
        x_flat, w1_w, w1_b.reshape(1, F), w2_w, w2_b.reshape(1, M))
    ebuf = _dispatch_fn()(x_flat, dest.reshape(N))
    mlp_out = _mlp_call(ebuf, mlp1, mlp2)
    gath = _combine_fn()(mlp_out, safe.reshape(N))
    out = _select_call(gath, x_flat, maskf)
    return out.reshape(b, s, d), logits.reshape(b, s, M)
